# baseline (device time: 30386 ns/iter reference)
import jax
import jax.numpy as jnp
from jax import lax
from jax.experimental import pallas as pl
from jax.experimental.pallas import tpu as pltpu

N_DEV = 4


def kernel(x, w_mat):
    m_per, k = x.shape
    _, n_per = w_mat.shape

    def body(x_ref, w_ref, out_ref, comm_ref, send_sems, recv_sems):
        my = lax.axis_index("i")
        left = lax.rem(my - 1 + N_DEV, N_DEV)
        right = lax.rem(my + 1, N_DEV)

        barrier_sem = pltpu.get_barrier_semaphore()
        for nbr in (left, right):
            pl.semaphore_signal(
                barrier_sem, inc=1,
                device_id=(nbr,), device_id_type=pl.DeviceIdType.MESH,
            )
        pl.semaphore_wait(barrier_sem, 2)

        w = w_ref[...].astype(jnp.bfloat16)

        comm_ref[0] = x_ref[...].astype(jnp.bfloat16)
        out_ref[pl.ds(my * m_per, m_per), :] = jnp.maximum(
            jnp.dot(comm_ref[0], w, preferred_element_type=jnp.float32), 0.0
        )

        for h in range(N_DEV - 1):
            rdma = pltpu.make_async_remote_copy(
                src_ref=comm_ref.at[h],
                dst_ref=comm_ref.at[h + 1],
                send_sem=send_sems.at[h],
                recv_sem=recv_sems.at[h],
                device_id=(right,),
                device_id_type=pl.DeviceIdType.MESH,
            )
            rdma.start()
            rdma.wait()

            origin = lax.rem(my - h - 1 + N_DEV, N_DEV)
            out_ref[pl.ds(origin * m_per, m_per), :] = jnp.maximum(
                jnp.dot(comm_ref[h + 1], w, preferred_element_type=jnp.float32),
                0.0,
            )

    return pl.pallas_call(
        body,
        out_shape=jax.ShapeDtypeStruct((N_DEV * m_per, n_per), jnp.float32),
        in_specs=[
            pl.BlockSpec(memory_space=pltpu.VMEM),
            pl.BlockSpec(memory_space=pltpu.VMEM),
        ],
        out_specs=pl.BlockSpec(memory_space=pltpu.VMEM),
        scratch_shapes=[
            pltpu.VMEM((N_DEV, m_per, k), jnp.bfloat16),
            pltpu.SemaphoreType.DMA((N_DEV - 1,)),
            pltpu.SemaphoreType.DMA((N_DEV - 1,)),
        ],
        compiler_params=pltpu.CompilerParams(collective_id=0),
    )(x, w_mat)


# device time: 16374 ns/iter; 1.8557x vs baseline; 1.8557x over previous
import jax
import jax.numpy as jnp
from jax import lax
from jax.experimental import pallas as pl
from jax.experimental.pallas import tpu as pltpu

N_DEV = 4


def kernel(x, w_mat):
    m_per, k = x.shape
    _, n_per = w_mat.shape
    m_half = m_per // 2

    def body(x_ref, w_ref, out_ref, xbf_ref, l_ref, r_ref,
             send_sems, l_sems, r_sems):
        my = lax.axis_index("i")
        left = lax.rem(my + N_DEV - 1, N_DEV)
        right = lax.rem(my + 1, N_DEV)
        opp = lax.rem(my + 2, N_DEV)

        barrier_sem = pltpu.get_barrier_semaphore()
        for nbr in (left, right):
            pl.semaphore_signal(
                barrier_sem, inc=1,
                device_id=(nbr,), device_id_type=pl.DeviceIdType.MESH,
            )
        pl.semaphore_wait(barrier_sem, 2)

        xbf_ref[0] = x_ref[0:m_half, :].astype(jnp.bfloat16)
        xbf_ref[1] = x_ref[m_half:m_per, :].astype(jnp.bfloat16)

        def copy(src, dst, ssem, rsem, dev):
            return pltpu.make_async_remote_copy(
                src_ref=src, dst_ref=dst, send_sem=ssem, recv_sem=rsem,
                device_id=(dev,), device_id_type=pl.DeviceIdType.MESH,
            )

        s_t_r = copy(xbf_ref.at[0], l_ref.at[0], send_sems.at[0],
                     l_sems.at[0], right)
        s_b_l = copy(xbf_ref.at[1], r_ref.at[1], send_sems.at[1],
                     r_sems.at[1], left)
        s_b_r = copy(xbf_ref.at[1], l_ref.at[1], send_sems.at[2],
                     l_sems.at[1], right)
        s_t_l = copy(xbf_ref.at[0], r_ref.at[0], send_sems.at[3],
                     r_sems.at[0], left)
        s_t_r.start()
        s_b_l.start()
        s_b_r.start()
        s_t_l.start()

        w = w_ref[...].astype(jnp.bfloat16)

        def gemm(src_val, row_start):
            out_ref[pl.ds(row_start, src_val.shape[0]), :] = jnp.maximum(
                jnp.dot(src_val, w, preferred_element_type=jnp.float32), 0.0
            )

        gemm(jnp.concatenate([xbf_ref[0], xbf_ref[1]], axis=0), my * m_per)

        s_t_r.wait_recv()
        f_t = copy(l_ref.at[0], l_ref.at[2], send_sems.at[4],
                   l_sems.at[2], right)
        f_t.start()
        s_b_l.wait_recv()
        f_b = copy(r_ref.at[1], r_ref.at[2], send_sems.at[5],
                   r_sems.at[2], left)
        f_b.start()

        gemm(l_ref[0], left * m_per)
        gemm(r_ref[1], right * m_per + m_half)

        s_b_r.wait_recv()
        gemm(l_ref[1], left * m_per + m_half)
        s_t_l.wait_recv()
        gemm(r_ref[0], right * m_per)

        f_t.wait_recv()
        gemm(l_ref[2], opp * m_per)
        f_b.wait_recv()
        gemm(r_ref[2], opp * m_per + m_half)

        for s in (s_t_r, s_b_l, s_b_r, s_t_l, f_t, f_b):
            s.wait_send()

    return pl.pallas_call(
        body,
        out_shape=jax.ShapeDtypeStruct((N_DEV * m_per, n_per), jnp.float32),
        in_specs=[
            pl.BlockSpec(memory_space=pltpu.VMEM),
            pl.BlockSpec(memory_space=pltpu.VMEM),
        ],
        out_specs=pl.BlockSpec(memory_space=pltpu.VMEM),
        scratch_shapes=[
            pltpu.VMEM((2, m_half, k), jnp.bfloat16),
            pltpu.VMEM((3, m_half, k), jnp.bfloat16),
            pltpu.VMEM((3, m_half, k), jnp.bfloat16),
            pltpu.SemaphoreType.DMA((6,)),
            pltpu.SemaphoreType.DMA((3,)),
            pltpu.SemaphoreType.DMA((3,)),
        ],
        compiler_params=pltpu.CompilerParams(collective_id=0),
    )(x, w_mat)
